# baseline (speedup 1.0000x reference)
import functools

import jax
import jax.numpy as jnp
from jax import lax
from jax.experimental import pallas as pl
from jax.experimental.pallas import tpu as pltpu
from jax.experimental.pallas import tpu_sc as plsc

_NC, _NS = 2, 16
_NW = _NC * _NS
_B, _S, _D = 4, 4096, 1024
_SEQ_PER_W = _S // _NW
_CS = 8
_NJ = _SEQ_PER_W // _CS
_R = _B * _CS
_L = 16
_CPR = _D // _L
_SCALE = 32.0
_UNROLL = 8


def _body(xc_hbm, table_hbm, pe_hbm, out_hbm,
          idx_v, pe0, pe1, rows0, rows1, rows2, g0, g1, g2, p0, p1,
          o0, o1, o2):
    wid = lax.axis_index("s") * _NC + lax.axis_index("c")
    s_base = wid * _SEQ_PER_W
    rows = (rows0, rows1, rows2)
    pe = (pe0, pe1)
    gsem = (g0, g1, g2)
    psem = (p0, p1)
    osem = (o0, o1, o2)

    pltpu.sync_copy(xc_hbm.at[wid], idx_v)

    def start_gather(t):
        p = t % 3
        return [pltpu.async_copy(table_hbm.at[idx_v.at[t]], rows[p], gsem[p])]

    def start_pe(t):
        return pltpu.async_copy(
            pe_hbm.at[pl.ds(s_base + t * _CS, _CS)], pe[t % 2], psem[t % 2])

    g_cp = {0: start_gather(0), 1: start_gather(1)}
    pe_cp = {0: start_pe(0), 1: start_pe(1)}
    o_cp = {}

    for t in range(_NJ):
        p = t % 3
        if t + 2 < _NJ:
            if t - 1 >= 0:
                for cp in o_cp[t - 1]:
                    cp.wait()
            g_cp[t + 2] = start_gather(t + 2)
        pe_cp[t].wait()
        for cp in g_cp[t]:
            cp.wait()

        rows_p, pe_p = rows[p], pe[t % 2]

        s0 = s_base + t * _CS
        o_cp[t] = []
        for b in range(_B):
            @plsc.parallel_loop(0, _CS * _CPR, unroll=_UNROLL)
            def _fma(e, b=b):
                r = e // _CPR
                sl = pl.ds((e % _CPR) * _L, _L)
                rows_p[b * _CS + r, sl] = (
                    rows_p[b * _CS + r, sl] * _SCALE + pe_p[r, sl])

            o_cp[t].append(
                pltpu.async_copy(rows_p.at[pl.ds(b * _CS, _CS)],
                                 out_hbm.at[b, pl.ds(s0, _CS)], osem[p]))
        if t + 2 < _NJ:
            pe_cp[t + 2] = start_pe(t + 2)

    for t in (_NJ - 3, _NJ - 2, _NJ - 1):
        for cp in o_cp[t]:
            cp.wait()


@jax.jit
def kernel(x, embedding_table, positional_encoding):
    xc = (x.astype(jnp.int32)
          .reshape(_B, _NW, _NJ, _CS)
          .transpose(1, 2, 0, 3)
          .reshape(_NW, _NJ, _R))
    run = pl.kernel(
        _body,
        out_type=jax.ShapeDtypeStruct((_B, _S, _D), jnp.float32),
        mesh=plsc.VectorSubcoreMesh(core_axis_name="c", subcore_axis_name="s"),
        scratch_types=[
            pltpu.VMEM((_NJ, _R), jnp.int32),
            pltpu.VMEM((_CS, _D), jnp.float32),
            pltpu.VMEM((_CS, _D), jnp.float32),
            pltpu.VMEM((_R, _D), jnp.float32),
            pltpu.VMEM((_R, _D), jnp.float32),
            pltpu.VMEM((_R, _D), jnp.float32),
            pltpu.SemaphoreType.DMA,
            pltpu.SemaphoreType.DMA,
            pltpu.SemaphoreType.DMA,
            pltpu.SemaphoreType.DMA,
            pltpu.SemaphoreType.DMA,
            pltpu.SemaphoreType.DMA,
            pltpu.SemaphoreType.DMA,
            pltpu.SemaphoreType.DMA,
        ],
    )
    return run(xc, embedding_table, positional_encoding)


# speedup vs baseline: 1.0507x; 1.0507x over previous
import functools

import jax
import jax.numpy as jnp
from jax import lax
from jax.experimental import pallas as pl
from jax.experimental.pallas import tpu as pltpu
from jax.experimental.pallas import tpu_sc as plsc

_NC, _NS = 2, 16
_NW = _NC * _NS
_B, _S, _D = 4, 4096, 1024
_SEQ_PER_W = _S // _NW
_CS = 8
_NJ = _SEQ_PER_W // _CS
_R = _B * _CS
_L = 16
_CPR = _D // _L
_SCALE = 32.0
_UNROLL = 8


def _body(xc_hbm, table_hbm, pe_hbm, out_hbm,
          idx_v, pe0, pe1, rows0, rows1, rows2, g0, g1, g2, p0, p1,
          o0, o1, o2):
    wid = lax.axis_index("s") * _NC + lax.axis_index("c")
    s_base = wid * _SEQ_PER_W
    rows = (rows0, rows1, rows2)
    pe = (pe0, pe1)
    gsem = (g0, g1, g2)
    psem = (p0, p1)
    osem = (o0, o1, o2)

    pltpu.sync_copy(xc_hbm.at[wid], idx_v)

    def start_gather(t):
        p = t % 3
        return [pltpu.async_copy(table_hbm.at[idx_v.at[t]], rows[p], gsem[p])]

    def start_pe(t):
        return pltpu.async_copy(
            pe_hbm.at[pl.ds(s_base + t * _CS, _CS)], pe[t % 2], psem[t % 2])

    g_cp = {0: start_gather(0), 1: start_gather(1)}
    pe_cp = {0: start_pe(0), 1: start_pe(1)}
    o_cp = {}

    for t in range(_NJ):
        p = t % 3
        if t + 2 < _NJ:
            if t - 1 >= 0:
                for cp in o_cp[t - 1]:
                    cp.wait()
            g_cp[t + 2] = start_gather(t + 2)
        pe_cp[t].wait()
        for cp in g_cp[t]:
            cp.wait()

        rows_p, pe_p = rows[p], pe[t % 2]

        @plsc.parallel_loop(0, _CS * _CPR, unroll=_UNROLL)
        def _fma(e):
            r = e // _CPR
            sl = pl.ds((e % _CPR) * _L, _L)
            pe_reg = pe_p[r, sl]
            for b in range(_B):
                rows_p[b * _CS + r, sl] = rows_p[b * _CS + r, sl] * _SCALE + pe_reg

        s0 = s_base + t * _CS
        o_cp[t] = [
            pltpu.async_copy(rows_p.at[pl.ds(b * _CS, _CS)],
                             out_hbm.at[b, pl.ds(s0, _CS)], osem[p])
            for b in range(_B)
        ]
        if t + 2 < _NJ:
            pe_cp[t + 2] = start_pe(t + 2)

    for t in (_NJ - 3, _NJ - 2, _NJ - 1):
        for cp in o_cp[t]:
            cp.wait()


@jax.jit
def kernel(x, embedding_table, positional_encoding):
    xc = (x.astype(jnp.int32)
          .reshape(_B, _NW, _NJ, _CS)
          .transpose(1, 2, 0, 3)
          .reshape(_NW, _NJ, _R))
    run = pl.kernel(
        _body,
        out_type=jax.ShapeDtypeStruct((_B, _S, _D), jnp.float32),
        mesh=plsc.VectorSubcoreMesh(core_axis_name="c", subcore_axis_name="s"),
        scratch_types=[
            pltpu.VMEM((_NJ, _R), jnp.int32),
            pltpu.VMEM((_CS, _D), jnp.float32),
            pltpu.VMEM((_CS, _D), jnp.float32),
            pltpu.VMEM((_R, _D), jnp.float32),
            pltpu.VMEM((_R, _D), jnp.float32),
            pltpu.VMEM((_R, _D), jnp.float32),
            pltpu.SemaphoreType.DMA,
            pltpu.SemaphoreType.DMA,
            pltpu.SemaphoreType.DMA,
            pltpu.SemaphoreType.DMA,
            pltpu.SemaphoreType.DMA,
            pltpu.SemaphoreType.DMA,
            pltpu.SemaphoreType.DMA,
            pltpu.SemaphoreType.DMA,
        ],
    )
    return run(xc, embedding_table, positional_encoding)


# speedup vs baseline: 1.0511x; 1.0004x over previous
import jax
import jax.numpy as jnp
from jax import lax
from jax.experimental import pallas as pl
from jax.experimental.pallas import tpu as pltpu
from jax.experimental.pallas import tpu_sc as plsc

_NC, _NS = 2, 16
_NW = _NC * _NS
_B, _S, _D = 4, 4096, 1024
_SEQ_PER_W = _S // _NW
_CS = 8
_NJ = _SEQ_PER_W // _CS
_R = _B * _CS
_L = 16
_CPR = _D // _L
_SCALE = 32.0
_UNROLL = 8


def _body(xc_hbm, table_hbm, pe_hbm, out_hbm,
          idx_v, pe0, pe1, rows0, rows1, rows2, g0, g1, g2, p0, p1,
          o0, o1, o2):
    wid = lax.axis_index("s") * _NC + lax.axis_index("c")
    s_base = wid * _SEQ_PER_W
    rows = (rows0, rows1, rows2)
    pe = (pe0, pe1)
    gsem = (g0, g1, g2)
    psem = (p0, p1)
    osem = (o0, o1, o2)

    pltpu.sync_copy(xc_hbm.at[wid], idx_v)

    def start_gather(t):
        p = t % 3
        return [pltpu.async_copy(table_hbm.at[idx_v.at[t]], rows[p], gsem[p])]

    def start_pe(t):
        return pltpu.async_copy(
            pe_hbm.at[pl.ds(s_base + t * _CS, _CS)], pe[t % 2], psem[t % 2])

    g_cp = {0: start_gather(0), 1: start_gather(1)}
    pe_cp = {0: start_pe(0), 1: start_pe(1)}
    o_cp = {}

    for t in range(_NJ):
        p = t % 3
        if t + 2 < _NJ:
            if t - 1 >= 0:
                for cp in o_cp[t - 1]:
                    cp.wait()
            g_cp[t + 2] = start_gather(t + 2)
        pe_cp[t].wait()
        for cp in g_cp[t]:
            cp.wait()

        rows_p, pe_p = rows[p], pe[t % 2]

        @plsc.parallel_loop(0, _CS * _CPR, unroll=_UNROLL)
        def _fma(e):
            r = e // _CPR
            sl = pl.ds((e % _CPR) * _L, _L)
            pe_reg = pe_p[r, sl]
            for b in range(_B):
                rows_p[b * _CS + r, sl] = rows_p[b * _CS + r, sl] * _SCALE + pe_reg

        s0 = s_base + t * _CS
        o_cp[t] = [
            pltpu.async_copy(rows_p.at[pl.ds(b * _CS, _CS)],
                             out_hbm.at[b, pl.ds(s0, _CS)], osem[p])
            for b in range(_B)
        ]
        if t + 2 < _NJ:
            pe_cp[t + 2] = start_pe(t + 2)

    for t in (_NJ - 3, _NJ - 2, _NJ - 1):
        for cp in o_cp[t]:
            cp.wait()


@jax.jit
def kernel(x, embedding_table, positional_encoding):
    xc = (x.astype(jnp.int32)
          .reshape(_B, _NW, _NJ, _CS)
          .transpose(1, 2, 0, 3)
          .reshape(_NW, _NJ, _R))
    run = pl.kernel(
        _body,
        out_type=jax.ShapeDtypeStruct((_B, _S, _D), jnp.float32),
        mesh=plsc.VectorSubcoreMesh(core_axis_name="c", subcore_axis_name="s"),
        scratch_types=[
            pltpu.VMEM((_NJ, _R), jnp.int32),
            pltpu.VMEM((_CS, _D), jnp.float32),
            pltpu.VMEM((_CS, _D), jnp.float32),
            pltpu.VMEM((_R, _D), jnp.float32),
            pltpu.VMEM((_R, _D), jnp.float32),
            pltpu.VMEM((_R, _D), jnp.float32),
            pltpu.SemaphoreType.DMA,
            pltpu.SemaphoreType.DMA,
            pltpu.SemaphoreType.DMA,
            pltpu.SemaphoreType.DMA,
            pltpu.SemaphoreType.DMA,
            pltpu.SemaphoreType.DMA,
            pltpu.SemaphoreType.DMA,
            pltpu.SemaphoreType.DMA,
        ],
    )
    return run(xc, embedding_table, positional_encoding)
